# baseline (device time: 43871 ns/iter reference)
import jax
import jax.numpy as jnp
from jax import lax
from jax.experimental import pallas as pl
from jax.experimental.pallas import tpu as pltpu

B, S, H, Dh, Dr = 2, 256, 16, 64, 32
D = 1024
DC_SH = 64
F32 = jnp.float32


def kernel(x, Wdkv, Wuk, Wuv, Wq, Wqr, Wkr, Wo):
    def body(
        x_ref, wdkv_ref, wuk_ref, wuv_ref, wq_ref, wqr_ref, wkr_ref, wo_ref,
        out_ref,
        c_ref, c_peer_ref, wuk_peer_ref, wuv_peer_ref, o_ref,
        send_sems, recv_sems,
    ):
        my_x = lax.axis_index("x")
        my_y = lax.axis_index("y")
        my_z = lax.axis_index("z")
        peer = (my_x, my_y, 1 - my_z)

        barrier_sem = pltpu.get_barrier_semaphore()
        pl.semaphore_signal(
            barrier_sem, inc=1, device_id=peer,
            device_id_type=pl.DeviceIdType.MESH,
        )
        pl.semaphore_wait(barrier_sem, 1)

        rdma_wuk = pltpu.make_async_remote_copy(
            src_ref=wuk_ref, dst_ref=wuk_peer_ref,
            send_sem=send_sems.at[0], recv_sem=recv_sems.at[0],
            device_id=peer, device_id_type=pl.DeviceIdType.MESH,
        )
        rdma_wuk.start()
        rdma_wuv = pltpu.make_async_remote_copy(
            src_ref=wuv_ref, dst_ref=wuv_peer_ref,
            send_sem=send_sems.at[1], recv_sem=recv_sems.at[1],
            device_id=peer, device_id_type=pl.DeviceIdType.MESH,
        )
        rdma_wuv.start()

        for b in range(B):
            c_ref[b] = jnp.dot(x_ref[b], wdkv_ref[...], preferred_element_type=F32)
        rdma_c = pltpu.make_async_remote_copy(
            src_ref=c_ref, dst_ref=c_peer_ref,
            send_sem=send_sems.at[2], recv_sem=recv_sems.at[2],
            device_id=peer, device_id_type=pl.DeviceIdType.MESH,
        )
        rdma_c.start()

        qs, qrs, krs = [], [], []
        for b in range(B):
            xb = x_ref[b]
            qs.append(jnp.dot(xb, wq_ref[...], preferred_element_type=F32))
            qrs.append(jnp.dot(xb, wqr_ref[...], preferred_element_type=F32))
            krs.append(jnp.dot(xb, wkr_ref[...], preferred_element_type=F32))

        rdma_wuk.wait()
        rdma_wuv.wait()
        rdma_c.wait()

        scale = (Dh + Dr) ** -0.5
        dn = (((1,), (1,)), ((), ()))
        for b in range(B):
            kb = (
                jnp.dot(c_ref[b], wuk_ref[...], preferred_element_type=F32)
                + jnp.dot(c_peer_ref[b], wuk_peer_ref[...], preferred_element_type=F32)
            )
            vb = (
                jnp.dot(c_ref[b], wuv_ref[...], preferred_element_type=F32)
                + jnp.dot(c_peer_ref[b], wuv_peer_ref[...], preferred_element_type=F32)
            )
            qb, qrb, krb = qs[b], qrs[b], krs[b]
            for h in range(H):
                q = qb[:, h * Dh:(h + 1) * Dh]
                k = kb[:, h * Dh:(h + 1) * Dh]
                qr = qrb[:, h * Dr:(h + 1) * Dr]
                s = (
                    lax.dot_general(q, k, dn, preferred_element_type=F32)
                    + lax.dot_general(qr, krb, dn, preferred_element_type=F32)
                ) * scale
                m = jnp.max(s, axis=-1, keepdims=True)
                p = jnp.exp(s - m)
                p = p / jnp.sum(p, axis=-1, keepdims=True)
                o_ref[b, :, h * Dh:(h + 1) * Dh] = jnp.dot(
                    p, vb[:, h * Dh:(h + 1) * Dh], preferred_element_type=F32
                )
            out_ref[b] = jnp.dot(o_ref[b], wo_ref[...], preferred_element_type=F32)

    vmem = pl.BlockSpec(memory_space=pltpu.VMEM)
    return pl.pallas_call(
        body,
        out_shape=jax.ShapeDtypeStruct((B, S, D), F32),
        in_specs=[vmem] * 8,
        out_specs=vmem,
        scratch_shapes=[
            pltpu.VMEM((B, S, DC_SH), F32),
            pltpu.VMEM((B, S, DC_SH), F32),
            pltpu.VMEM((DC_SH, D), F32),
            pltpu.VMEM((DC_SH, D), F32),
            pltpu.VMEM((B, S, H * Dh), F32),
            pltpu.SemaphoreType.DMA((3,)),
            pltpu.SemaphoreType.DMA((3,)),
        ],
        compiler_params=pltpu.CompilerParams(collective_id=0),
    )(x, Wdkv, Wuk, Wuv, Wq, Wqr, Wkr, Wo)
